# baseline (speedup 1.0000x reference)
import functools

import jax
import jax.numpy as jnp
from jax import lax
from jax.experimental import pallas as pl
from jax.experimental.pallas import tpu as pltpu
from jax.experimental.pallas import tpu_sc as plsc

_SCORE_T = 0.01
_NEG = -1e9
_NEGH = -5e8
_MD = 100
_B = 8
_C = 8
_N = 5000
_NP = 5008
_CH = _NP // 16
_OP = 128
_RTP = 15040
_KP = 112
_TMP = 336


def _body(scores_t, bx, rot_p, trans_p,
          o_boxes, o_scores, o_labels, o_rot, o_trans,
          bxall_v, sc0_v, sc1_v,
          cs_v, cidx_v, tm_v, kx1_v, ky1_v, kx2_v, ky2_v, kar_v,
          res_s, res_i, sh_s, sh_i,
          ms_v, mi_v, os_v, ol_v, li_v, ob_v, or_v, ot_v,
          rt_v, tr_v, sem_b, sem_s0, sem_s1, sem_r, sem_t):
    ci = lax.axis_index("c")
    s = lax.axis_index("s")
    img = 4 * ci + s // 4
    cls0 = 2 * (s % 4)
    is_merger = s % 4 == 0

    bx_dma = pltpu.async_copy(bx.at[img], bxall_v, sem_b)
    sc0_dma = pltpu.async_copy(scores_t.at[img, cls0], sc0_v, sem_s0)
    sc1_dma = pltpu.async_copy(scores_t.at[img, cls0 + 1], sc1_v, sem_s1)

    @pl.when(is_merger)
    def _prefetch():
        pltpu.async_copy(rot_p.at[img], rt_v, sem_r)
        pltpu.async_copy(trans_p.at[img], tr_v, sem_t)

    iota16 = lax.iota(jnp.int32, 16)
    neg16 = jnp.full((16,), _NEG, jnp.float32)
    zero16i = jnp.zeros((16,), jnp.int32)
    big16i = jnp.full((16,), 2 ** 30, jnp.int32)
    ninf16 = jnp.full((16,), -jnp.inf, jnp.float32)
    lane0 = iota16 == 0

    def _full_i(v):
        return jnp.full((16,), v, jnp.int32)

    def init_body(i, carry):
        sl = pl.ds(i * 16, 16)
        res_s[0, sl] = neg16
        res_s[1, sl] = neg16
        res_i[0, sl] = zero16i
        res_i[1, sl] = zero16i
        return carry
    lax.fori_loop(0, _OP // 16, init_body, 0)

    bx_dma.wait()
    for p in range(2):
        sc_v = sc0_v if p == 0 else sc1_v
        (sc0_dma if p == 0 else sc1_dma).wait()

        def kinit(i, carry):
            sl = pl.ds(i * 16, 16)
            kx1_v[sl] = jnp.full((16,), 3e9, jnp.float32)
            ky1_v[sl] = jnp.full((16,), 3e9, jnp.float32)
            kx2_v[sl] = jnp.zeros((16,), jnp.float32)
            ky2_v[sl] = jnp.zeros((16,), jnp.float32)
            kar_v[sl] = jnp.zeros((16,), jnp.float32)
            return carry
        lax.fori_loop(0, _KP // 16, kinit, 0)

        def band_step(t, nk):
            band = 15 - t

            def do_band(nk):
                def comp_body(i, off16):
                    sl = pl.ds(i * 16, 16)
                    v = sc_v[sl]
                    bb = jnp.clip(v * 16.0, 0.0, 15.0).astype(jnp.int32)
                    m = (v > _SCORE_T) & (bb == band)
                    pc = plsc.cumsum(m.astype(jnp.int32))
                    posv = off16 + pc - 1
                    plsc.store_scatter(cs_v, [posv], v, mask=m)
                    plsc.store_scatter(cidx_v, [posv], i * 16 + iota16,
                                       mask=m)
                    return off16 + plsc.all_reduce_population_count(m)
                off16 = lax.fori_loop(0, _CH, comp_body, zero16i)
                nc_cand = jnp.max(off16)
                ncch = (nc_cand + 15) // 16
                padm = (nc_cand + iota16) < ncch * 16
                plsc.store_scatter(cs_v, [nc_cand + iota16], neg16, mask=padm)

                def tm_body(j, carry):
                    v = cs_v[pl.ds(j * 16, 16)]
                    plsc.store_scatter(tm_v, [_full_i(j)],
                                       jnp.full((16,), jnp.max(v)),
                                       mask=lane0)
                    return carry
                lax.fori_loop(0, ncch, tm_body, 0)
                ntch = (ncch + 15) // 16
                padm2 = (ncch + iota16) < ntch * 16
                plsc.store_scatter(tm_v, [ncch + iota16], ninf16, mask=padm2)

                def walk_body(e, nk):
                    def do_cand(nk):
                        def tms(j, c):
                            b0, bi = c
                            v = tm_v[pl.ds(j * 16, 16)]
                            m = v > b0
                            return (jnp.where(m, v, b0),
                                    jnp.where(m, _full_i(j), bi))
                        b0, bi = lax.fori_loop(0, ntch, tms,
                                               (ninf16, zero16i))
                        bmax = jnp.max(b0)
                        jstar = jnp.min(jnp.where(b0 == bmax,
                                                  bi * 16 + iota16, big16i))
                        v = cs_v[pl.ds(jstar * 16, 16)]
                        lminv = plsc.all_reduce_ffs(v == bmax)
                        pos16 = _full_i(jstar * 16) + lminv
                        plsc.store_scatter(cs_v, [pos16], neg16, mask=lane0)
                        newm = jnp.max(jnp.where(iota16 == lminv, neg16, v))
                        plsc.store_scatter(tm_v, [_full_i(jstar)],
                                           jnp.full((16,), newm), mask=lane0)
                        idx16 = plsc.load_gather(cidx_v, [pos16])
                        bx1 = plsc.load_gather(bxall_v, [zero16i, idx16])
                        by1 = plsc.load_gather(bxall_v, [_full_i(1), idx16])
                        bx2 = plsc.load_gather(bxall_v, [_full_i(2), idx16])
                        by2 = plsc.load_gather(bxall_v, [_full_i(3), idx16])
                        bar = (bx2 - bx1) * (by2 - by1)

                        supv = jnp.zeros((16,), jnp.bool_)
                        for t7 in range(_KP // 16):
                            sl = pl.ds(t7 * 16, 16)
                            xx1 = jnp.maximum(bx1, kx1_v[sl])
                            yy1 = jnp.maximum(by1, ky1_v[sl])
                            xx2 = jnp.minimum(bx2, kx2_v[sl])
                            yy2 = jnp.minimum(by2, ky2_v[sl])
                            inter = (jnp.maximum(xx2 - xx1, 0.0)
                                     * jnp.maximum(yy2 - yy1, 0.0))
                            union = kar_v[sl] + bar - inter
                            supv = supv | (inter + inter > union)
                        sup = jnp.any(supv)

                        keepm = lane0 & jnp.full((16,),
                                                 jnp.logical_not(sup))
                        nk16 = _full_i(nk)
                        plsc.store_scatter(kx1_v, [nk16], bx1, mask=keepm)
                        plsc.store_scatter(ky1_v, [nk16], by1, mask=keepm)
                        plsc.store_scatter(kx2_v, [nk16], bx2, mask=keepm)
                        plsc.store_scatter(ky2_v, [nk16], by2, mask=keepm)
                        plsc.store_scatter(kar_v, [nk16], bar, mask=keepm)
                        plsc.store_scatter(res_s, [_full_i(p), nk16],
                                           jnp.full((16,), bmax, jnp.float32),
                                           mask=keepm)
                        plsc.store_scatter(res_i, [_full_i(p), nk16], idx16,
                                           mask=keepm)
                        return nk + jnp.where(sup, 0, 1)
                    return lax.cond(nk < _MD, do_cand, lambda n: n, nk)
                return lax.fori_loop(0, nc_cand, walk_body, nk)
            return lax.cond(nk < _MD, do_band, lambda n: n, nk)
        lax.fori_loop(0, 16, band_step, jnp.int32(0))

    pltpu.sync_copy(res_s, sh_s.at[s])
    pltpu.sync_copy(res_i, sh_i.at[s])
    plsc.subcore_barrier()

    @pl.when(is_merger)
    def _merge():
        pltpu.make_async_copy(rot_p.at[img], rt_v, sem_r).wait()
        pltpu.make_async_copy(trans_p.at[img], tr_v, sem_t).wait()
        q = s // 4
        ms_dma = pltpu.async_copy(sh_s.at[pl.ds(4 * q, 4)], ms_v, sem_s0)
        mi_dma = pltpu.async_copy(sh_i.at[pl.ds(4 * q, 4)], mi_v, sem_s1)
        def oinit(i, carry):
            sl = pl.ds(i * 16, 16)
            os_v[sl] = neg16
            ol_v[sl] = zero16i
            li_v[sl] = zero16i
            return carry
        lax.fori_loop(0, _OP // 16, oinit, 0)
        ms_dma.wait()
        mi_dma.wait()

        tregs = []
        for r in range(4):
            def mtm(j, c, r=r):
                v = ms_v[r, (j // 8) % 2, pl.ds((j % 8) * 16, 16)]
                mx = jnp.full((16,), jnp.max(v))
                return jnp.where(iota16 == j, mx, c)
            tregs.append(lax.fori_loop(0, 16, mtm, ninf16))

        def m_step(k, carry):
            t0, t1, t2, t3 = carry
            bmax = jnp.max(jnp.maximum(jnp.maximum(t0, t1),
                                       jnp.maximum(t2, t3)))
            e0, e1, e2, e3 = (t0 == bmax), (t1 == bmax), (t2 == bmax), \
                (t3 == bmax)
            h0 = plsc.all_reduce_population_count(e0) > 0
            h1 = plsc.all_reduce_population_count(e1) > 0
            h2 = plsc.all_reduce_population_count(e2) > 0
            selm = jnp.where(h0, e0, jnp.where(h1, e1,
                                               jnp.where(h2, e2, e3)))
            vregi = jnp.where(h0, zero16i,
                              jnp.where(h1, _full_i(1),
                                        jnp.where(h2, _full_i(2),
                                                  _full_i(3))))
            jstar16 = vregi * 16 + plsc.all_reduce_ffs(selm)
            posv = jstar16 * 16 + iota16
            v = plsc.load_gather(ms_v, [posv // 256, (posv // 128) % 2,
                                        posv % 128])
            lminv = plsc.all_reduce_ffs(v == bmax)
            f16 = jstar16 * 16 + lminv
            k16 = _full_i(k)
            plsc.store_scatter(ms_v, [f16 // 256, (f16 // 128) % 2,
                                      f16 % 128], neg16, mask=lane0)
            newm16 = jnp.full((16,), jnp.max(jnp.where(iota16 == lminv,
                                                       neg16, v)))
            upd = iota16 == (jstar16 % 16)
            t0 = jnp.where(upd & (vregi == 0), newm16, t0)
            t1 = jnp.where(upd & (vregi == 1), newm16, t1)
            t2 = jnp.where(upd & (vregi == 2), newm16, t2)
            t3 = jnp.where(upd & (vregi == 3), newm16, t3)
            plsc.store_scatter(os_v, [k16],
                               jnp.full((16,), bmax, jnp.float32), mask=lane0)
            plsc.store_scatter(ol_v, [k16], f16 // _OP, mask=lane0)
            mi16 = plsc.load_gather(mi_v, [f16 // 256, (f16 // 128) % 2,
                                           f16 % 128])
            plsc.store_scatter(li_v, [k16], mi16, mask=lane0)
            return t0, t1, t2, t3
        lax.fori_loop(0, _MD, m_step, tuple(tregs))

        neg1 = jnp.full((16,), -1.0, jnp.float32)
        neg1i = jnp.full((16,), -1, jnp.int32)
        for t in range(_OP // 16):
            sl = pl.ds(t * 16, 16)
            sv = os_v[sl]
            val = sv > _NEGH
            os_v[sl] = jnp.where(val, sv, neg1)
            ol_v[sl] = jnp.where(val, ol_v[sl], neg1i)
            e = t * 16 + iota16
            idxv = li_v[sl]
            for comp in range(4):
                v = plsc.load_gather(bxall_v, [_full_i(comp), idxv])
                plsc.store_scatter(ob_v, [e, _full_i(comp)],
                                   jnp.where(val, v, neg1))
            idx3 = idxv * 3
            for comp in range(3):
                v = plsc.load_gather(rt_v, [idx3 + comp])
                plsc.store_scatter(or_v, [e, _full_i(comp)],
                                   jnp.where(val, v, neg1))
                w = plsc.load_gather(tr_v, [idx3 + comp])
                plsc.store_scatter(ot_v, [e, _full_i(comp)],
                                   jnp.where(val, w, neg1))

        d1 = pltpu.async_copy(ob_v, o_boxes.at[img], sem_b)
        d2 = pltpu.async_copy(os_v, o_scores.at[img], sem_s0)
        d3 = pltpu.async_copy(ol_v, o_labels.at[img], sem_s1)
        d4 = pltpu.async_copy(or_v, o_rot.at[img], sem_r)
        d5 = pltpu.async_copy(ot_v, o_trans.at[img], sem_t)
        d1.wait()
        d2.wait()
        d3.wait()
        d4.wait()
        d5.wait()


_sc_call = functools.partial(
    pl.kernel,
    out_type=[
        jax.ShapeDtypeStruct((_B, _OP, 4), jnp.float32),
        jax.ShapeDtypeStruct((_B, _OP), jnp.float32),
        jax.ShapeDtypeStruct((_B, _OP), jnp.int32),
        jax.ShapeDtypeStruct((_B, _OP, 3), jnp.float32),
        jax.ShapeDtypeStruct((_B, _OP, 3), jnp.float32),
    ],
    mesh=plsc.VectorSubcoreMesh(core_axis_name="c", subcore_axis_name="s",
                                num_cores=2, num_subcores=16),
    compiler_params=pltpu.CompilerParams(needs_layout_passes=False,
                                         use_tc_tiling_on_sc=False),
    scratch_types=[
        pltpu.VMEM((4, _NP), jnp.float32),
        pltpu.VMEM((_NP,), jnp.float32),
        pltpu.VMEM((_NP,), jnp.float32),
        pltpu.VMEM((_NP,), jnp.float32),
        pltpu.VMEM((_NP,), jnp.int32),
        pltpu.VMEM((_TMP,), jnp.float32),
        pltpu.VMEM((_KP,), jnp.float32),
        pltpu.VMEM((_KP,), jnp.float32),
        pltpu.VMEM((_KP,), jnp.float32),
        pltpu.VMEM((_KP,), jnp.float32),
        pltpu.VMEM((_KP,), jnp.float32),
        pltpu.VMEM((2, _OP), jnp.float32),
        pltpu.VMEM((2, _OP), jnp.int32),
        pltpu.VMEM_SHARED((16, 2, _OP), jnp.float32),
        pltpu.VMEM_SHARED((16, 2, _OP), jnp.int32),
        pltpu.VMEM((4, 2, _OP), jnp.float32),
        pltpu.VMEM((4, 2, _OP), jnp.int32),
        pltpu.VMEM((_OP,), jnp.float32),
        pltpu.VMEM((_OP,), jnp.int32),
        pltpu.VMEM((_OP,), jnp.int32),
        pltpu.VMEM((_OP, 4), jnp.float32),
        pltpu.VMEM((_OP, 3), jnp.float32),
        pltpu.VMEM((_OP, 3), jnp.float32),
        pltpu.VMEM((_RTP,), jnp.float32),
        pltpu.VMEM((_RTP,), jnp.float32),
        pltpu.SemaphoreType.DMA,
        pltpu.SemaphoreType.DMA,
        pltpu.SemaphoreType.DMA,
        pltpu.SemaphoreType.DMA,
        pltpu.SemaphoreType.DMA,
    ],
)(_body)


@jax.jit
def kernel(boxes, classification, rotation, translation):
    scores_t = jnp.pad(jnp.transpose(classification, (0, 2, 1)),
                       ((0, 0), (0, 0), (0, _NP - _N)),
                       constant_values=_NEG)
    bx = jnp.pad(jnp.transpose(boxes, (0, 2, 1)),
                 ((0, 0), (0, 0), (0, _NP - _N)))
    rot_p = jnp.pad(rotation.reshape(_B, 3 * _N),
                    ((0, 0), (0, _RTP - 3 * _N)))
    trans_p = jnp.pad(translation.reshape(_B, 3 * _N),
                      ((0, 0), (0, _RTP - 3 * _N)))
    ob, osc, ol, orr, otr = _sc_call(scores_t, bx, rot_p, trans_p)
    return (ob[:, :_MD], osc[:, :_MD], ol[:, :_MD],
            orr[:, :_MD], otr[:, :_MD])


# speedup vs baseline: 1.0451x; 1.0451x over previous
import functools

import jax
import jax.numpy as jnp
from jax import lax
from jax.experimental import pallas as pl
from jax.experimental.pallas import tpu as pltpu
from jax.experimental.pallas import tpu_sc as plsc

_SCORE_T = 0.01
_NEG = -1e9
_NEGH = -5e8
_MD = 100
_B = 8
_C = 8
_N = 5000
_NP = 5008
_CH = _NP // 16
_OP = 128
_RTP = 15040
_KP = 112
_TMP = 336


def _body(scores_t, bx, rot_p, trans_p,
          o_boxes, o_scores, o_labels, o_rot, o_trans,
          bxall_v, sc0_v, sc1_v,
          cs_v, cidx_v, tm_v, kx1_v, ky1_v, kx2_v, ky2_v, kar_v,
          res_s, res_i, sh_s, sh_i,
          ms_v, mi_v, os_v, ol_v, li_v, ob_v, or_v, ot_v,
          rt_v, tr_v, sem_b, sem_s0, sem_s1, sem_r, sem_t):
    ci = lax.axis_index("c")
    s = lax.axis_index("s")
    img = 4 * ci + s // 4
    cls0 = 2 * (s % 4)
    is_merger = s % 4 == 0

    bx_dma = pltpu.async_copy(bx.at[img], bxall_v, sem_b)
    sc0_dma = pltpu.async_copy(scores_t.at[img, cls0], sc0_v, sem_s0)
    sc1_dma = pltpu.async_copy(scores_t.at[img, cls0 + 1], sc1_v, sem_s1)

    @pl.when(is_merger)
    def _prefetch():
        pltpu.async_copy(rot_p.at[img], rt_v, sem_r)
        pltpu.async_copy(trans_p.at[img], tr_v, sem_t)

    iota16 = lax.iota(jnp.int32, 16)
    neg16 = jnp.full((16,), _NEG, jnp.float32)
    zero16i = jnp.zeros((16,), jnp.int32)
    big16i = jnp.full((16,), 2 ** 30, jnp.int32)
    ninf16 = jnp.full((16,), -jnp.inf, jnp.float32)
    lane0 = iota16 == 0

    def _full_i(v):
        return jnp.full((16,), v, jnp.int32)

    def init_body(i, carry):
        sl = pl.ds(i * 16, 16)
        res_s[0, sl] = neg16
        res_s[1, sl] = neg16
        res_i[0, sl] = zero16i
        res_i[1, sl] = zero16i
        return carry
    lax.fori_loop(0, _OP // 16, init_body, 0)

    bx_dma.wait()
    for p in range(2):
        sc_v = sc0_v if p == 0 else sc1_v
        (sc0_dma if p == 0 else sc1_dma).wait()

        def kinit(i, carry):
            sl = pl.ds(i * 16, 16)
            kx1_v[sl] = jnp.full((16,), 3e9, jnp.float32)
            ky1_v[sl] = jnp.full((16,), 3e9, jnp.float32)
            kx2_v[sl] = jnp.zeros((16,), jnp.float32)
            ky2_v[sl] = jnp.zeros((16,), jnp.float32)
            kar_v[sl] = jnp.zeros((16,), jnp.float32)
            return carry
        lax.fori_loop(0, _KP // 16, kinit, 0)

        def band_step(t, nk):
            band = 15 - t

            def do_band(nk):
                def comp_body(i, off16):
                    sl = pl.ds(i * 16, 16)
                    v = sc_v[sl]
                    bb = jnp.clip(v * 16.0, 0.0, 15.0).astype(jnp.int32)
                    m = (v > _SCORE_T) & (bb == band)
                    pc = plsc.cumsum(m.astype(jnp.int32))
                    posv = off16 + pc - 1
                    plsc.store_scatter(cs_v, [posv], v, mask=m)
                    plsc.store_scatter(cidx_v, [posv], i * 16 + iota16,
                                       mask=m)
                    return off16 + plsc.all_reduce_population_count(m)
                off16 = lax.fori_loop(0, _CH, comp_body, zero16i)
                nc_cand = jnp.max(off16)
                ncch = (nc_cand + 15) // 16
                padm = (nc_cand + iota16) < ncch * 16
                plsc.store_scatter(cs_v, [nc_cand + iota16], neg16, mask=padm)

                def test_keep(nk, pos16, bmax):
                    idx16 = plsc.load_gather(cidx_v, [pos16])
                    bx1 = plsc.load_gather(bxall_v, [zero16i, idx16])
                    by1 = plsc.load_gather(bxall_v, [_full_i(1), idx16])
                    bx2 = plsc.load_gather(bxall_v, [_full_i(2), idx16])
                    by2 = plsc.load_gather(bxall_v, [_full_i(3), idx16])
                    bar = (bx2 - bx1) * (by2 - by1)

                    supv = jnp.zeros((16,), jnp.bool_)
                    for t7 in range(_KP // 16):
                        sl = pl.ds(t7 * 16, 16)
                        xx1 = jnp.maximum(bx1, kx1_v[sl])
                        yy1 = jnp.maximum(by1, ky1_v[sl])
                        xx2 = jnp.minimum(bx2, kx2_v[sl])
                        yy2 = jnp.minimum(by2, ky2_v[sl])
                        inter = (jnp.maximum(xx2 - xx1, 0.0)
                                 * jnp.maximum(yy2 - yy1, 0.0))
                        union = kar_v[sl] + bar - inter
                        supv = supv | (inter + inter > union)
                    sup = jnp.any(supv)

                    keepm = lane0 & jnp.full((16,), jnp.logical_not(sup))
                    nk16 = _full_i(nk)
                    plsc.store_scatter(kx1_v, [nk16], bx1, mask=keepm)
                    plsc.store_scatter(ky1_v, [nk16], by1, mask=keepm)
                    plsc.store_scatter(kx2_v, [nk16], bx2, mask=keepm)
                    plsc.store_scatter(ky2_v, [nk16], by2, mask=keepm)
                    plsc.store_scatter(kar_v, [nk16], bar, mask=keepm)
                    plsc.store_scatter(res_s, [_full_i(p), nk16],
                                       jnp.full((16,), bmax, jnp.float32),
                                       mask=keepm)
                    plsc.store_scatter(res_i, [_full_i(p), nk16], idx16,
                                       mask=keepm)
                    return nk + jnp.where(sup, 0, 1)

                def walk_small(nk):
                    uregs = []
                    for r in range(2):
                        def bld(j, c, r=r):
                            v = cs_v[pl.ds((16 * r + j) * 16, 16)]
                            mx = jnp.where(16 * r + j < ncch,
                                           jnp.max(v), -jnp.inf)
                            return jnp.where(iota16 == j,
                                             jnp.full((16,), mx), c)
                        uregs.append(lax.fori_loop(0, 16, bld, ninf16))

                    def walk_body(e, carry):
                        def do_cand(carry):
                            nk, u0, u1 = carry
                            bmax = jnp.max(jnp.maximum(u0, u1))
                            e0 = u0 == bmax
                            e1 = u1 == bmax
                            h0 = plsc.all_reduce_population_count(e0) > 0
                            selm = jnp.where(h0, e0, e1)
                            vregi = jnp.where(h0, zero16i, _full_i(1))
                            jstar16 = vregi * 16 + plsc.all_reduce_ffs(selm)
                            posv = jstar16 * 16 + iota16
                            v = plsc.load_gather(cs_v, [posv])
                            lminv = plsc.all_reduce_ffs(v == bmax)
                            pos16 = jstar16 * 16 + lminv
                            plsc.store_scatter(cs_v, [pos16], neg16,
                                               mask=lane0)
                            newm16 = jnp.full(
                                (16,), jnp.max(jnp.where(iota16 == lminv,
                                                         neg16, v)))
                            upd = iota16 == (jstar16 % 16)
                            u0 = jnp.where(upd & (vregi == 0), newm16, u0)
                            u1 = jnp.where(upd & (vregi == 1), newm16, u1)
                            return test_keep(nk, pos16, bmax), u0, u1
                        nk, _, _ = carry
                        return lax.cond(nk < _MD, do_cand, lambda c: c, carry)
                    out = lax.fori_loop(0, nc_cand, walk_body,
                                        (nk, uregs[0], uregs[1]))
                    return out[0]

                def walk_large(nk):
                    def tm_body(j, carry):
                        v = cs_v[pl.ds(j * 16, 16)]
                        plsc.store_scatter(tm_v, [_full_i(j)],
                                           jnp.full((16,), jnp.max(v)),
                                           mask=lane0)
                        return carry
                    lax.fori_loop(0, ncch, tm_body, 0)
                    ntch = (ncch + 15) // 16
                    padm2 = (ncch + iota16) < ntch * 16
                    plsc.store_scatter(tm_v, [ncch + iota16], ninf16,
                                       mask=padm2)

                    def walk_body(e, nk):
                        def do_cand(nk):
                            def tms(j, c):
                                b0, bi = c
                                v = tm_v[pl.ds(j * 16, 16)]
                                m = v > b0
                                return (jnp.where(m, v, b0),
                                        jnp.where(m, _full_i(j), bi))
                            b0, bi = lax.fori_loop(0, ntch, tms,
                                                   (ninf16, zero16i))
                            bmax = jnp.max(b0)
                            jstar = jnp.min(jnp.where(b0 == bmax,
                                                      bi * 16 + iota16,
                                                      big16i))
                            v = cs_v[pl.ds(jstar * 16, 16)]
                            lminv = plsc.all_reduce_ffs(v == bmax)
                            pos16 = _full_i(jstar * 16) + lminv
                            plsc.store_scatter(cs_v, [pos16], neg16,
                                               mask=lane0)
                            newm = jnp.max(jnp.where(iota16 == lminv,
                                                     neg16, v))
                            plsc.store_scatter(tm_v, [_full_i(jstar)],
                                               jnp.full((16,), newm),
                                               mask=lane0)
                            return test_keep(nk, pos16, bmax)
                        return lax.cond(nk < _MD, do_cand, lambda n: n, nk)
                    return lax.fori_loop(0, nc_cand, walk_body, nk)

                return lax.cond(nc_cand <= 512, walk_small, walk_large, nk)
            return lax.cond(nk < _MD, do_band, lambda n: n, nk)
        lax.fori_loop(0, 16, band_step, jnp.int32(0))

    pltpu.sync_copy(res_s, sh_s.at[s])
    pltpu.sync_copy(res_i, sh_i.at[s])
    plsc.subcore_barrier()

    @pl.when(is_merger)
    def _merge():
        pltpu.make_async_copy(rot_p.at[img], rt_v, sem_r).wait()
        pltpu.make_async_copy(trans_p.at[img], tr_v, sem_t).wait()
        q = s // 4
        ms_dma = pltpu.async_copy(sh_s.at[pl.ds(4 * q, 4)], ms_v, sem_s0)
        mi_dma = pltpu.async_copy(sh_i.at[pl.ds(4 * q, 4)], mi_v, sem_s1)
        def oinit(i, carry):
            sl = pl.ds(i * 16, 16)
            os_v[sl] = neg16
            ol_v[sl] = zero16i
            li_v[sl] = zero16i
            return carry
        lax.fori_loop(0, _OP // 16, oinit, 0)
        ms_dma.wait()
        mi_dma.wait()

        tregs = []
        for r in range(4):
            def mtm(j, c, r=r):
                v = ms_v[r, (j // 8) % 2, pl.ds((j % 8) * 16, 16)]
                mx = jnp.full((16,), jnp.max(v))
                return jnp.where(iota16 == j, mx, c)
            tregs.append(lax.fori_loop(0, 16, mtm, ninf16))

        def m_step(k, carry):
            t0, t1, t2, t3 = carry
            bmax = jnp.max(jnp.maximum(jnp.maximum(t0, t1),
                                       jnp.maximum(t2, t3)))
            e0, e1, e2, e3 = (t0 == bmax), (t1 == bmax), (t2 == bmax), \
                (t3 == bmax)
            h0 = plsc.all_reduce_population_count(e0) > 0
            h1 = plsc.all_reduce_population_count(e1) > 0
            h2 = plsc.all_reduce_population_count(e2) > 0
            selm = jnp.where(h0, e0, jnp.where(h1, e1,
                                               jnp.where(h2, e2, e3)))
            vregi = jnp.where(h0, zero16i,
                              jnp.where(h1, _full_i(1),
                                        jnp.where(h2, _full_i(2),
                                                  _full_i(3))))
            jstar16 = vregi * 16 + plsc.all_reduce_ffs(selm)
            posv = jstar16 * 16 + iota16
            v = plsc.load_gather(ms_v, [posv // 256, (posv // 128) % 2,
                                        posv % 128])
            lminv = plsc.all_reduce_ffs(v == bmax)
            f16 = jstar16 * 16 + lminv
            k16 = _full_i(k)
            plsc.store_scatter(ms_v, [f16 // 256, (f16 // 128) % 2,
                                      f16 % 128], neg16, mask=lane0)
            newm16 = jnp.full((16,), jnp.max(jnp.where(iota16 == lminv,
                                                       neg16, v)))
            upd = iota16 == (jstar16 % 16)
            t0 = jnp.where(upd & (vregi == 0), newm16, t0)
            t1 = jnp.where(upd & (vregi == 1), newm16, t1)
            t2 = jnp.where(upd & (vregi == 2), newm16, t2)
            t3 = jnp.where(upd & (vregi == 3), newm16, t3)
            plsc.store_scatter(os_v, [k16],
                               jnp.full((16,), bmax, jnp.float32), mask=lane0)
            plsc.store_scatter(ol_v, [k16], f16 // _OP, mask=lane0)
            mi16 = plsc.load_gather(mi_v, [f16 // 256, (f16 // 128) % 2,
                                           f16 % 128])
            plsc.store_scatter(li_v, [k16], mi16, mask=lane0)
            return t0, t1, t2, t3
        lax.fori_loop(0, _MD, m_step, tuple(tregs))

        neg1 = jnp.full((16,), -1.0, jnp.float32)
        neg1i = jnp.full((16,), -1, jnp.int32)
        for t in range(_OP // 16):
            sl = pl.ds(t * 16, 16)
            sv = os_v[sl]
            val = sv > _NEGH
            os_v[sl] = jnp.where(val, sv, neg1)
            ol_v[sl] = jnp.where(val, ol_v[sl], neg1i)
            e = t * 16 + iota16
            idxv = li_v[sl]
            for comp in range(4):
                v = plsc.load_gather(bxall_v, [_full_i(comp), idxv])
                plsc.store_scatter(ob_v, [e, _full_i(comp)],
                                   jnp.where(val, v, neg1))
            idx3 = idxv * 3
            for comp in range(3):
                v = plsc.load_gather(rt_v, [idx3 + comp])
                plsc.store_scatter(or_v, [e, _full_i(comp)],
                                   jnp.where(val, v, neg1))
                w = plsc.load_gather(tr_v, [idx3 + comp])
                plsc.store_scatter(ot_v, [e, _full_i(comp)],
                                   jnp.where(val, w, neg1))

        d1 = pltpu.async_copy(ob_v, o_boxes.at[img], sem_b)
        d2 = pltpu.async_copy(os_v, o_scores.at[img], sem_s0)
        d3 = pltpu.async_copy(ol_v, o_labels.at[img], sem_s1)
        d4 = pltpu.async_copy(or_v, o_rot.at[img], sem_r)
        d5 = pltpu.async_copy(ot_v, o_trans.at[img], sem_t)
        d1.wait()
        d2.wait()
        d3.wait()
        d4.wait()
        d5.wait()


_sc_call = functools.partial(
    pl.kernel,
    out_type=[
        jax.ShapeDtypeStruct((_B, _OP, 4), jnp.float32),
        jax.ShapeDtypeStruct((_B, _OP), jnp.float32),
        jax.ShapeDtypeStruct((_B, _OP), jnp.int32),
        jax.ShapeDtypeStruct((_B, _OP, 3), jnp.float32),
        jax.ShapeDtypeStruct((_B, _OP, 3), jnp.float32),
    ],
    mesh=plsc.VectorSubcoreMesh(core_axis_name="c", subcore_axis_name="s",
                                num_cores=2, num_subcores=16),
    compiler_params=pltpu.CompilerParams(needs_layout_passes=False,
                                         use_tc_tiling_on_sc=False),
    scratch_types=[
        pltpu.VMEM((4, _NP), jnp.float32),
        pltpu.VMEM((_NP,), jnp.float32),
        pltpu.VMEM((_NP,), jnp.float32),
        pltpu.VMEM((_NP,), jnp.float32),
        pltpu.VMEM((_NP,), jnp.int32),
        pltpu.VMEM((_TMP,), jnp.float32),
        pltpu.VMEM((_KP,), jnp.float32),
        pltpu.VMEM((_KP,), jnp.float32),
        pltpu.VMEM((_KP,), jnp.float32),
        pltpu.VMEM((_KP,), jnp.float32),
        pltpu.VMEM((_KP,), jnp.float32),
        pltpu.VMEM((2, _OP), jnp.float32),
        pltpu.VMEM((2, _OP), jnp.int32),
        pltpu.VMEM_SHARED((16, 2, _OP), jnp.float32),
        pltpu.VMEM_SHARED((16, 2, _OP), jnp.int32),
        pltpu.VMEM((4, 2, _OP), jnp.float32),
        pltpu.VMEM((4, 2, _OP), jnp.int32),
        pltpu.VMEM((_OP,), jnp.float32),
        pltpu.VMEM((_OP,), jnp.int32),
        pltpu.VMEM((_OP,), jnp.int32),
        pltpu.VMEM((_OP, 4), jnp.float32),
        pltpu.VMEM((_OP, 3), jnp.float32),
        pltpu.VMEM((_OP, 3), jnp.float32),
        pltpu.VMEM((_RTP,), jnp.float32),
        pltpu.VMEM((_RTP,), jnp.float32),
        pltpu.SemaphoreType.DMA,
        pltpu.SemaphoreType.DMA,
        pltpu.SemaphoreType.DMA,
        pltpu.SemaphoreType.DMA,
        pltpu.SemaphoreType.DMA,
    ],
)(_body)


@jax.jit
def kernel(boxes, classification, rotation, translation):
    scores_t = jnp.pad(jnp.transpose(classification, (0, 2, 1)),
                       ((0, 0), (0, 0), (0, _NP - _N)),
                       constant_values=_NEG)
    bx = jnp.pad(jnp.transpose(boxes, (0, 2, 1)),
                 ((0, 0), (0, 0), (0, _NP - _N)))
    rot_p = jnp.pad(rotation.reshape(_B, 3 * _N),
                    ((0, 0), (0, _RTP - 3 * _N)))
    trans_p = jnp.pad(translation.reshape(_B, 3 * _N),
                      ((0, 0), (0, _RTP - 3 * _N)))
    ob, osc, ol, orr, otr = _sc_call(scores_t, bx, rot_p, trans_p)
    return (ob[:, :_MD], osc[:, :_MD], ol[:, :_MD],
            orr[:, :_MD], otr[:, :_MD])
